# baseline (speedup 1.0000x reference)
import functools

import jax
import jax.numpy as jnp
from jax import lax
from jax.experimental import pallas as pl
from jax.experimental.pallas import tpu as pltpu
from jax.experimental.pallas import tpu_sc as plsc

TS_START = 26
N_TABLE = 126
N_REP = 20
N_TS = N_TABLE - TS_START
F_OUT = TS_START + N_TS * N_REP
DIM = 64
W = F_OUT * DIM
N_CHUNK = 4
CH = W // N_CHUNK
GROUPS = CH // 16
NW = 32
B_PER_W = 1024 // NW


def _bias_kernel(table_ref, full_ref):
    full_ref[0:TS_START] = table_ref[0:TS_START]
    ts = table_ref[TS_START:N_TABLE]
    for r in range(N_REP):
        base = TS_START + r * N_TS
        full_ref[base:base + N_TS] = ts


def _sc_add(x_hbm, bias_hbm, out_hbm, buf, bias_buf):
    wid = lax.axis_index("s") * 2 + lax.axis_index("c")
    base = wid * B_PER_W
    for c in range(N_CHUNK):
        pltpu.sync_copy(bias_hbm.at[c], bias_buf)

        def bbody(bb, carry, c=c):
            b = base + bb
            pltpu.sync_copy(x_hbm.at[b, c], buf)

            def gbody(g, carry2):
                sl = pl.ds(g * 16, 16)
                buf[sl] = buf[sl] + bias_buf[sl]
                return carry2

            lax.fori_loop(0, GROUPS, gbody, 0)
            pltpu.sync_copy(buf, out_hbm.at[b, c])
            return carry

        lax.fori_loop(0, B_PER_W, bbody, 0)


def kernel(X, table):
    B = X.shape[0]
    full2d = pl.pallas_call(
        _bias_kernel,
        out_shape=jax.ShapeDtypeStruct((F_OUT, DIM), table.dtype),
    )(table)
    bias4 = full2d.reshape(N_CHUNK, CH)
    X4 = X.reshape(B, N_CHUNK, CH)
    sc_fn = functools.partial(
        pl.kernel,
        out_type=jax.ShapeDtypeStruct((B, N_CHUNK, CH), X.dtype),
        mesh=plsc.VectorSubcoreMesh(core_axis_name="c", subcore_axis_name="s"),
        scratch_types=[
            pltpu.VMEM((CH,), jnp.float32),
            pltpu.VMEM((CH,), jnp.float32),
        ],
    )(_sc_add)
    out = sc_fn(X4, bias4)
    return out.reshape(B, F_OUT, DIM)


# speedup vs baseline: 1.3702x; 1.3702x over previous
import functools

import jax
import jax.numpy as jnp
from jax import lax
from jax.experimental import pallas as pl
from jax.experimental.pallas import tpu as pltpu
from jax.experimental.pallas import tpu_sc as plsc

TS_START = 26
N_TABLE = 126
N_REP = 20
N_TS = N_TABLE - TS_START
F_OUT = TS_START + N_TS * N_REP
DIM = 64
W = F_OUT * DIM
N_CHUNK = 8
CH = W // N_CHUNK
GROUPS = CH // 16
NW = 32
B_PER_W = 1024 // NW


def _bias_kernel(table_ref, full_ref):
    full_ref[0:TS_START] = table_ref[0:TS_START]
    ts = table_ref[TS_START:N_TABLE]
    for r in range(N_REP):
        base = TS_START + r * N_TS
        full_ref[base:base + N_TS] = ts


def _sc_add(x_hbm, bias_hbm, out_hbm,
            in0, in1, ot0, ot1, bias_buf, si0, si1, so0, so1):
    in_bufs = (in0, in1)
    out_bufs = (ot0, ot1)
    in_sems = (si0, si1)
    out_sems = (so0, so1)
    wid = lax.axis_index("s") * 2 + lax.axis_index("c")
    base = wid * B_PER_W

    def in_copy(b, c, s):
        return pltpu.make_async_copy(x_hbm.at[b, c], in_bufs[s], in_sems[s])

    def out_copy(b, c, s):
        return pltpu.make_async_copy(out_bufs[s], out_hbm.at[b, c], out_sems[s])

    for c in range(N_CHUNK):
        pltpu.sync_copy(bias_hbm.at[c], bias_buf)
        in_copy(base, c, 0).start()
        in_copy(base + 1, c, 1).start()

        def kbody(k, carry, c=c):
            for s in range(2):
                bb = k * 2 + s
                b = base + bb
                in_copy(b, c, s).wait()

                @pl.when(bb >= 2)
                def _wait_out():
                    out_copy(b - 2, c, s).wait()

                def gbody(g, carry2, s=s):
                    sl = pl.ds(g * 16, 16)
                    out_bufs[s][sl] = in_bufs[s][sl] + bias_buf[sl]
                    return carry2

                lax.fori_loop(0, GROUPS, gbody, 0)
                out_copy(b, c, s).start()

                @pl.when(bb + 2 < B_PER_W)
                def _next_in():
                    in_copy(b + 2, c, s).start()

            return carry

        lax.fori_loop(0, B_PER_W // 2, kbody, 0)
        out_copy(base + B_PER_W - 2, c, 0).wait()
        out_copy(base + B_PER_W - 1, c, 1).wait()


def kernel(X, table):
    B = X.shape[0]
    full2d = pl.pallas_call(
        _bias_kernel,
        out_shape=jax.ShapeDtypeStruct((F_OUT, DIM), table.dtype),
    )(table)
    bias4 = full2d.reshape(N_CHUNK, CH)
    X4 = X.reshape(B, N_CHUNK, CH)
    sc_fn = functools.partial(
        pl.kernel,
        out_type=jax.ShapeDtypeStruct((B, N_CHUNK, CH), X.dtype),
        mesh=plsc.VectorSubcoreMesh(core_axis_name="c", subcore_axis_name="s"),
        scratch_types=[
            pltpu.VMEM((CH,), jnp.float32),
            pltpu.VMEM((CH,), jnp.float32),
            pltpu.VMEM((CH,), jnp.float32),
            pltpu.VMEM((CH,), jnp.float32),
            pltpu.VMEM((CH,), jnp.float32),
            pltpu.SemaphoreType.DMA,
            pltpu.SemaphoreType.DMA,
            pltpu.SemaphoreType.DMA,
            pltpu.SemaphoreType.DMA,
        ],
    )(_sc_add)
    out = sc_fn(X4, bias4)
    return out.reshape(B, F_OUT, DIM)


# speedup vs baseline: 2.0097x; 1.4667x over previous
import functools

import jax
import jax.numpy as jnp
from jax import lax
from jax.experimental import pallas as pl
from jax.experimental.pallas import tpu as pltpu
from jax.experimental.pallas import tpu_sc as plsc

TS_START = 26
N_TABLE = 126
N_REP = 20
N_TS = N_TABLE - TS_START
F_OUT = TS_START + N_TS * N_REP
DIM = 64
W = F_OUT * DIM
N_CHUNK = 8
CH = W // N_CHUNK
GROUPS = CH // 16
NW = 32
B_PER_W = 1024 // NW


def _bias_kernel(table_ref, full_ref):
    full_ref[0:TS_START] = table_ref[0:TS_START]
    ts = table_ref[TS_START:N_TABLE]
    for r in range(N_REP):
        base = TS_START + r * N_TS
        full_ref[base:base + N_TS] = ts


def _sc_add(x_hbm, bias_hbm, out_hbm,
            in0, in1, ot0, ot1, bias_buf, si0, si1, so0, so1):
    in_bufs = (in0, in1)
    out_bufs = (ot0, ot1)
    in_sems = (si0, si1)
    out_sems = (so0, so1)
    wid = lax.axis_index("s") * 2 + lax.axis_index("c")
    base = wid * B_PER_W

    def in_copy(b, c, s):
        return pltpu.make_async_copy(x_hbm.at[b, c], in_bufs[s], in_sems[s])

    def out_copy(b, c, s):
        return pltpu.make_async_copy(out_bufs[s], out_hbm.at[b, c], out_sems[s])

    for c in range(N_CHUNK):
        pltpu.sync_copy(bias_hbm.at[c], bias_buf)
        in_copy(base, c, 0).start()
        in_copy(base + 1, c, 1).start()

        def kbody(k, carry, c=c):
            for s in range(2):
                bb = k * 2 + s
                b = base + bb
                in_copy(b, c, s).wait()

                @pl.when(bb >= 2)
                def _wait_out():
                    out_copy(b - 2, c, s).wait()

                def _make_adds(s):
                    def _adds(i):
                        sl = pl.ds(i, 16)
                        out_bufs[s][sl] = in_bufs[s][sl] + bias_buf[sl]
                    return _adds

                plsc.parallel_loop(0, CH, 16, unroll=8)(_make_adds(s))
                out_copy(b, c, s).start()

                @pl.when(bb + 2 < B_PER_W)
                def _next_in():
                    in_copy(b + 2, c, s).start()

            return carry

        lax.fori_loop(0, B_PER_W // 2, kbody, 0)
        out_copy(base + B_PER_W - 2, c, 0).wait()
        out_copy(base + B_PER_W - 1, c, 1).wait()


def kernel(X, table):
    B = X.shape[0]
    full2d = pl.pallas_call(
        _bias_kernel,
        out_shape=jax.ShapeDtypeStruct((F_OUT, DIM), table.dtype),
    )(table)
    bias4 = full2d.reshape(N_CHUNK, CH)
    X4 = X.reshape(B, N_CHUNK, CH)
    sc_fn = functools.partial(
        pl.kernel,
        out_type=jax.ShapeDtypeStruct((B, N_CHUNK, CH), X.dtype),
        mesh=plsc.VectorSubcoreMesh(core_axis_name="c", subcore_axis_name="s"),
        scratch_types=[
            pltpu.VMEM((CH,), jnp.float32),
            pltpu.VMEM((CH,), jnp.float32),
            pltpu.VMEM((CH,), jnp.float32),
            pltpu.VMEM((CH,), jnp.float32),
            pltpu.VMEM((CH,), jnp.float32),
            pltpu.SemaphoreType.DMA,
            pltpu.SemaphoreType.DMA,
            pltpu.SemaphoreType.DMA,
            pltpu.SemaphoreType.DMA,
        ],
    )(_sc_add)
    out = sc_fn(X4, bias4)
    return out.reshape(B, F_OUT, DIM)


# speedup vs baseline: 2.0115x; 1.0009x over previous
import functools

import jax
import jax.numpy as jnp
from jax import lax
from jax.experimental import pallas as pl
from jax.experimental.pallas import tpu as pltpu
from jax.experimental.pallas import tpu_sc as plsc

TS_START = 26
N_TABLE = 126
N_REP = 20
N_TS = N_TABLE - TS_START
F_OUT = TS_START + N_TS * N_REP
DIM = 64
W = F_OUT * DIM
N_CHUNK = 8
CH = W // N_CHUNK
GROUPS = CH // 16
NW = 32
B_PER_W = 1024 // NW


def _bias_kernel(table_ref, full_ref):
    full_ref[0:TS_START] = table_ref[0:TS_START]
    ts = table_ref[TS_START:N_TABLE]
    for r in range(N_REP):
        base = TS_START + r * N_TS
        full_ref[base:base + N_TS] = ts


def _sc_add(x_hbm, bias_hbm, out_hbm,
            in0, in1, ot0, ot1, bias_buf, si0, si1, so0, so1):
    in_bufs = (in0, in1)
    out_bufs = (ot0, ot1)
    in_sems = (si0, si1)
    out_sems = (so0, so1)
    wid = lax.axis_index("s") * 2 + lax.axis_index("c")
    base = wid * B_PER_W

    def in_copy(b, c, s):
        return pltpu.make_async_copy(x_hbm.at[b, c], in_bufs[s], in_sems[s])

    def out_copy(b, c, s):
        return pltpu.make_async_copy(out_bufs[s], out_hbm.at[b, c], out_sems[s])

    for c in range(N_CHUNK):
        pltpu.sync_copy(bias_hbm.at[c], bias_buf)
        in_copy(base, c, 0).start()
        in_copy(base + 1, c, 1).start()

        def kbody(k, carry, c=c):
            for s in range(2):
                bb = k * 2 + s
                b = base + bb
                in_copy(b, c, s).wait()

                @pl.when(bb >= 2)
                def _wait_out():
                    out_copy(b - 2, c, s).wait()

                def _make_adds(s):
                    def _adds(i):
                        sl = pl.ds(i, 16)
                        out_bufs[s][sl] = in_bufs[s][sl] + bias_buf[sl]
                    return _adds

                plsc.parallel_loop(0, CH, 16, unroll=16)(_make_adds(s))
                out_copy(b, c, s).start()

                @pl.when(bb + 2 < B_PER_W)
                def _next_in():
                    in_copy(b + 2, c, s).start()

            return carry

        lax.fori_loop(0, B_PER_W // 2, kbody, 0)
        out_copy(base + B_PER_W - 2, c, 0).wait()
        out_copy(base + B_PER_W - 1, c, 1).wait()


def kernel(X, table):
    B = X.shape[0]
    full2d = pl.pallas_call(
        _bias_kernel,
        out_shape=jax.ShapeDtypeStruct((F_OUT, DIM), table.dtype),
    )(table)
    bias4 = full2d.reshape(N_CHUNK, CH)
    X4 = X.reshape(B, N_CHUNK, CH)
    sc_fn = functools.partial(
        pl.kernel,
        out_type=jax.ShapeDtypeStruct((B, N_CHUNK, CH), X.dtype),
        mesh=plsc.VectorSubcoreMesh(core_axis_name="c", subcore_axis_name="s"),
        scratch_types=[
            pltpu.VMEM((CH,), jnp.float32),
            pltpu.VMEM((CH,), jnp.float32),
            pltpu.VMEM((CH,), jnp.float32),
            pltpu.VMEM((CH,), jnp.float32),
            pltpu.VMEM((CH,), jnp.float32),
            pltpu.SemaphoreType.DMA,
            pltpu.SemaphoreType.DMA,
            pltpu.SemaphoreType.DMA,
            pltpu.SemaphoreType.DMA,
        ],
    )(_sc_add)
    out = sc_fn(X4, bias4)
    return out.reshape(B, F_OUT, DIM)
